# Initial kernel scaffold; baseline (speedup 1.0000x reference)
import jax
import jax.numpy as jnp
from jax.experimental import pallas as pl


def kernel(x, edge_index, W, a):
    raise NotImplementedError("write your pallas kernel here")


# speedup vs baseline: 21.6547x; 21.6547x over previous
import functools

import jax
import jax.numpy as jnp
from jax import lax
from jax.experimental import pallas as pl
from jax.experimental.pallas import tpu as pltpu
from jax.experimental.pallas import tpu_sc as plsc

N = 10000
E = 320000
F_IN = 128
HID = 32
HEADS = 4
ALPHA = 0.2

FEAT = HEADS * HID

_GATHER_DNUMS = lax.GatherDimensionNumbers(
    offset_dims=(), collapsed_slice_dims=(0,), start_index_map=(0,))


def _splat(vec, idx):
    return lax.gather(vec, idx[:, None], _GATHER_DNUMS, (1,),
                      mode=lax.GatherScatterMode.PROMISE_IN_BOUNDS)
ROW = FEAT + 16
CHUNK = 128
NC, NS = 2, 16
NW = NC * NS
TOTAL_CHUNKS = E // CHUNK
MAX_CHUNKS_PER_W = -(-TOTAL_CHUNKS // NW)
ROWS_PER_TILE = N // NS


def _prep_body(x_ref, wcat_ref, a1_ref, a2_ref, tbl_ref, s1n_ref):
    h = jnp.dot(x_ref[...], wcat_ref[...],
                preferred_element_type=jnp.float32,
                precision=lax.Precision.HIGHEST)
    s2 = jnp.dot(h, a2_ref[...],
                 preferred_element_type=jnp.float32,
                 precision=lax.Precision.HIGHEST)
    pad = jnp.zeros((h.shape[0], ROW - FEAT - HEADS), jnp.float32)
    tbl_ref[...] = jnp.concatenate([h, s2, pad], axis=1)
    s1 = jnp.dot(h, a1_ref[...],
                 preferred_element_type=jnp.float32,
                 precision=lax.Precision.HIGHEST)
    pad1 = jnp.zeros((h.shape[0], 16 - HEADS), jnp.float32)
    s1n_ref[...] = jnp.concatenate([s1, pad1], axis=1)


def _edge_body(tbl_hbm, s1_hbm, ei_hbm, out_hbm,
               acc_sh, s1b, gbuf, srcb, dstb, sem, sem2):
    cid = lax.axis_index("c")
    sid = lax.axis_index("s")
    wid = sid * NC + cid

    zero16 = jnp.zeros((16,), jnp.float32)

    @pl.loop(0, CHUNK)
    def _(r):
        @pl.loop(0, ROW, step=16)
        def _(c):
            gbuf[r, pl.ds(c, 16)] = zero16

    row0 = sid * ROWS_PER_TILE
    nfull = ROWS_PER_TILE // CHUNK
    rem = ROWS_PER_TILE - nfull * CHUNK

    @pl.loop(0, nfull)
    def _(i):
        pltpu.sync_copy(gbuf, acc_sh.at[pl.ds(row0 + i * CHUNK, CHUNK)])

    pltpu.sync_copy(gbuf.at[pl.ds(0, rem)],
                    acc_sh.at[pl.ds(row0 + nfull * CHUNK, rem)])
    plsc.subcore_barrier()

    lanes = lax.iota(jnp.int32, 16)

    @pl.loop(0, MAX_CHUNKS_PER_W)
    def _(j):
        chunk_id = wid + j * NW

        @pl.when(chunk_id < TOTAL_CHUNKS)
        def _():
            ebase = chunk_id * CHUNK
            pltpu.sync_copy(ei_hbm.at[0, pl.ds(ebase, CHUNK)], srcb)
            pltpu.sync_copy(ei_hbm.at[1, pl.ds(ebase, CHUNK)], dstb)
            cp1 = pltpu.async_copy(tbl_hbm.at[dstb], gbuf, sem)
            cp2 = pltpu.async_copy(s1_hbm.at[srcb], s1b, sem2)
            cp1.wait()
            cp2.wait()

            @pl.loop(0, CHUNK, step=16)
            def _(g):
                rowids = g + lanes
                evs = []
                for h in range(HEADS):
                    s1v = plsc.load_gather(
                        s1b, [rowids, jnp.full((16,), h, jnp.int32)])
                    s2v = plsc.load_gather(
                        gbuf, [rowids, jnp.full((16,), FEAT + h, jnp.int32)])
                    z = s1v + s2v
                    evs.append(jnp.exp(-jnp.maximum(z, ALPHA * z)))
                for li in range(16):
                    i = g + li
                    sel = jnp.full((16,), li, jnp.int32)
                    tail = zero16
                    for h in range(HEADS):
                        evv = _splat(evs[h], sel)
                        for k in range(HID // 16):
                            c0 = h * HID + k * 16
                            gbuf[i, pl.ds(c0, 16)] = (
                                gbuf[i, pl.ds(c0, 16)] * evv)
                        tail = jnp.where(lanes == h, evv, tail)
                    gbuf[i, pl.ds(FEAT, 16)] = tail

            pltpu.sync_copy(gbuf, acc_sh.at[srcb], add=True)

    plsc.subcore_barrier()

    @pl.loop(0, nfull)
    def _(i):
        pltpu.sync_copy(acc_sh.at[pl.ds(row0 + i * CHUNK, CHUNK)],
                        out_hbm.at[cid, pl.ds(row0 + i * CHUNK, CHUNK)])

    pltpu.sync_copy(acc_sh.at[pl.ds(row0 + nfull * CHUNK, rem)],
                    out_hbm.at[cid, pl.ds(row0 + nfull * CHUNK, rem)])


def _fin_body(acc_ref, out_ref):
    a = acc_ref[0] + acc_ref[1]
    feats = a[:, :FEAT]
    rows = a[:, FEAT:FEAT + HEADS]
    denom = jnp.concatenate(
        [jnp.broadcast_to(rows[:, h:h + 1], (a.shape[0], HID))
         for h in range(HEADS)], axis=1)
    hp = jnp.where(denom > 0.0, feats / denom, 0.0)

    def elu(v):
        return jnp.where(v > 0.0, v, jnp.exp(jnp.minimum(v, 0.0)) - 1.0)

    out_ref[...] = elu(elu(hp))


@jax.jit
def kernel(x, edge_index, W, a):
    wcat = W.transpose(1, 0, 2).reshape(F_IN, FEAT)
    a1 = a[:, 0, :HID]
    a2 = a[:, 0, HID:]
    eyeh = jnp.eye(HEADS, dtype=jnp.float32)
    A1 = (a1[:, :, None] * eyeh[:, None, :]).reshape(FEAT, HEADS)
    A2 = (a2[:, :, None] * eyeh[:, None, :]).reshape(FEAT, HEADS)

    tbl, s1n = pl.pallas_call(
        _prep_body,
        out_shape=(
            jax.ShapeDtypeStruct((N, ROW), jnp.float32),
            jax.ShapeDtypeStruct((N, 16), jnp.float32),
        ),
    )(x, wcat, A1, A2)

    mesh = plsc.VectorSubcoreMesh(core_axis_name="c", subcore_axis_name="s")
    edge_kernel = functools.partial(
        pl.kernel,
        out_type=jax.ShapeDtypeStruct((NC, N, ROW), jnp.float32),
        mesh=mesh,
        scratch_types=[
            pltpu.VMEM_SHARED((N, ROW), jnp.float32),
            pltpu.VMEM((CHUNK, 16), jnp.float32),
            pltpu.VMEM((CHUNK, ROW), jnp.float32),
            pltpu.VMEM((CHUNK,), jnp.int32),
            pltpu.VMEM((CHUNK,), jnp.int32),
            pltpu.SemaphoreType.DMA,
            pltpu.SemaphoreType.DMA,
        ],
        compiler_params=pltpu.CompilerParams(
            use_tc_tiling_on_sc=False, needs_layout_passes=False),
    )(_edge_body)
    acc = edge_kernel(tbl, s1n, edge_index)

    out = pl.pallas_call(
        _fin_body,
        out_shape=jax.ShapeDtypeStruct((N, FEAT), jnp.float32),
    )(acc)
    return out


# speedup vs baseline: 29.0385x; 1.3410x over previous
import functools

import jax
import jax.numpy as jnp
from jax import lax
from jax.experimental import pallas as pl
from jax.experimental.pallas import tpu as pltpu
from jax.experimental.pallas import tpu_sc as plsc

N = 10000
E = 320000
F_IN = 128
HID = 32
HEADS = 4
ALPHA = 0.2

FEAT = HEADS * HID
ROW = FEAT + 16

_GATHER_DNUMS = lax.GatherDimensionNumbers(
    offset_dims=(), collapsed_slice_dims=(0,), start_index_map=(0,))


def _splat(vec, idx):
    return lax.gather(vec, idx[:, None], _GATHER_DNUMS, (1,),
                      mode=lax.GatherScatterMode.PROMISE_IN_BOUNDS)
CHUNK = 128
NC, NS = 2, 16
NW = NC * NS
TOTAL_CHUNKS = E // CHUNK
MAX_CHUNKS_PER_W = -(-TOTAL_CHUNKS // NW)
ROWS_PER_TILE = N // NS


def _prep_body(x_ref, wcat_ref, a1_ref, a2_ref, tbl_ref, s1n_ref):
    h = jnp.dot(x_ref[...], wcat_ref[...],
                preferred_element_type=jnp.float32,
                precision=lax.Precision.HIGHEST)
    s2 = jnp.dot(h, a2_ref[...],
                 preferred_element_type=jnp.float32,
                 precision=lax.Precision.HIGHEST)
    pad = jnp.zeros((h.shape[0], ROW - FEAT - HEADS), jnp.float32)
    tbl_ref[...] = jnp.concatenate([h, s2, pad], axis=1)
    s1 = jnp.dot(h, a1_ref[...],
                 preferred_element_type=jnp.float32,
                 precision=lax.Precision.HIGHEST)
    pad1 = jnp.zeros((h.shape[0], 8 - HEADS), jnp.float32)
    s1n_ref[...] = jnp.concatenate([s1, pad1], axis=1)


def _edge_body(tbl_hbm, s1_hbm, ei_hbm, out_hbm,
               acc_sh, s1b0, s1b1, gbuf0, gbuf1,
               srcb0, srcb1, dstb0, dstb1, ebuf,
               semg0, semg1, sems0, sems1):
    cid = lax.axis_index("c")
    sid = lax.axis_index("s")
    wid = sid * NC + cid
    s1bs = (s1b0, s1b1)
    gbufs = (gbuf0, gbuf1)
    srcbs = (srcb0, srcb1)
    dstbs = (dstb0, dstb1)
    semgs = (semg0, semg1)
    semss = (sems0, sems1)

    zero16 = jnp.zeros((16,), jnp.float32)

    @pl.loop(0, CHUNK)
    def _(r):
        @pl.loop(0, ROW, step=16)
        def _(c):
            gbuf0[r, pl.ds(c, 16)] = zero16

    row0 = sid * ROWS_PER_TILE
    nfull = ROWS_PER_TILE // CHUNK
    rem = ROWS_PER_TILE - nfull * CHUNK

    @pl.loop(0, nfull)
    def _(i):
        pltpu.sync_copy(gbuf0, acc_sh.at[pl.ds(row0 + i * CHUNK, CHUNK)])

    pltpu.sync_copy(gbuf0.at[pl.ds(0, rem)],
                    acc_sh.at[pl.ds(row0 + nfull * CHUNK, rem)])
    plsc.subcore_barrier()

    lanes = lax.iota(jnp.int32, 16)
    lanes_c = jnp.minimum(lanes, HEADS - 1)
    tailmask = lanes < HEADS

    def issue(t, b):
        @pl.when(t * NW + wid < TOTAL_CHUNKS)
        def _():
            ebase = (t * NW + wid) * CHUNK
            pltpu.sync_copy(ei_hbm.at[0, pl.ds(ebase, CHUNK)], srcbs[b])
            pltpu.sync_copy(ei_hbm.at[1, pl.ds(ebase, CHUNK)], dstbs[b])
            pltpu.async_copy(tbl_hbm.at[dstbs[b]], gbufs[b], semgs[b])
            pltpu.async_copy(s1_hbm.at[srcbs[b]], s1bs[b], semss[b])

    def wait(t, b):
        @pl.when(t * NW + wid < TOTAL_CHUNKS)
        def _():
            pltpu.make_async_copy(
                tbl_hbm.at[dstbs[b]], gbufs[b], semgs[b]).wait()
            pltpu.make_async_copy(
                s1_hbm.at[srcbs[b]], s1bs[b], semss[b]).wait()

    def compute(t, b):
        @pl.when(t * NW + wid < TOTAL_CHUNKS)
        def _():
            gbuf = gbufs[b]
            s1b = s1bs[b]

            @pl.loop(0, CHUNK, step=16)
            def _(g):
                rowids = g + lanes
                evs = []
                for h in range(HEADS):
                    s1v = plsc.load_gather(
                        s1b, [rowids, jnp.full((16,), h, jnp.int32)])
                    s2v = plsc.load_gather(
                        gbuf, [rowids, jnp.full((16,), FEAT + h, jnp.int32)])
                    z = s1v + s2v
                    evs.append(jnp.exp(-jnp.maximum(z, ALPHA * z)))
                for li in range(16):
                    i = g + li
                    sel = jnp.full((16,), li, jnp.int32)
                    tail = jnp.zeros((16,), jnp.float32)
                    for h in range(HEADS):
                        evv = _splat(evs[h], sel)
                        for k in range(HID // 16):
                            c0 = h * HID + k * 16
                            gbuf[i, pl.ds(c0, 16)] = (
                                gbuf[i, pl.ds(c0, 16)] * evv)
                        tail = jnp.where(lanes == h, evv, tail)
                    gbuf[i, pl.ds(FEAT, 16)] = tail

            pltpu.sync_copy(gbuf, acc_sh.at[srcbs[b]], add=True)

    issue(0, 0)
    npair = (MAX_CHUNKS_PER_W + 1) // 2

    @pl.loop(0, npair)
    def _(p):
        t0 = 2 * p
        wait(t0, 0)
        issue(t0 + 1, 1)
        compute(t0, 0)
        wait(t0 + 1, 1)
        issue(t0 + 2, 0)
        compute(t0 + 1, 1)

    plsc.subcore_barrier()

    @pl.loop(0, nfull)
    def _(i):
        pltpu.sync_copy(acc_sh.at[pl.ds(row0 + i * CHUNK, CHUNK)],
                        out_hbm.at[cid, pl.ds(row0 + i * CHUNK, CHUNK)])

    pltpu.sync_copy(acc_sh.at[pl.ds(row0 + nfull * CHUNK, rem)],
                    out_hbm.at[cid, pl.ds(row0 + nfull * CHUNK, rem)])


def _fin_body(acc_ref, out_ref):
    a = acc_ref[0] + acc_ref[1]
    feats = a[:, :FEAT]
    rows = a[:, FEAT:FEAT + HEADS]
    denom = jnp.concatenate(
        [jnp.broadcast_to(rows[:, h:h + 1], (a.shape[0], HID))
         for h in range(HEADS)], axis=1)
    hp = jnp.where(denom > 0.0, feats / denom, 0.0)

    def elu(v):
        return jnp.where(v > 0.0, v, jnp.exp(jnp.minimum(v, 0.0)) - 1.0)

    out_ref[...] = elu(elu(hp))


@jax.jit
def kernel(x, edge_index, W, a):
    wcat = W.transpose(1, 0, 2).reshape(F_IN, FEAT)
    a1 = a[:, 0, :HID]
    a2 = a[:, 0, HID:]
    eyeh = jnp.eye(HEADS, dtype=jnp.float32)
    A1 = (a1[:, :, None] * eyeh[:, None, :]).reshape(FEAT, HEADS)
    A2 = (a2[:, :, None] * eyeh[:, None, :]).reshape(FEAT, HEADS)

    tbl, s1n = pl.pallas_call(
        _prep_body,
        out_shape=(
            jax.ShapeDtypeStruct((N, ROW), jnp.float32),
            jax.ShapeDtypeStruct((N, 8), jnp.float32),
        ),
    )(x, wcat, A1, A2)

    mesh = plsc.VectorSubcoreMesh(core_axis_name="c", subcore_axis_name="s")
    edge_kernel = functools.partial(
        pl.kernel,
        out_type=jax.ShapeDtypeStruct((NC, N, ROW), jnp.float32),
        mesh=mesh,
        scratch_types=[
            pltpu.VMEM_SHARED((N, ROW), jnp.float32),
            pltpu.VMEM((CHUNK, 8), jnp.float32),
            pltpu.VMEM((CHUNK, 8), jnp.float32),
            pltpu.VMEM((CHUNK, ROW), jnp.float32),
            pltpu.VMEM((CHUNK, ROW), jnp.float32),
            pltpu.VMEM((CHUNK,), jnp.int32),
            pltpu.VMEM((CHUNK,), jnp.int32),
            pltpu.VMEM((CHUNK,), jnp.int32),
            pltpu.VMEM((CHUNK,), jnp.int32),
            pltpu.VMEM((HEADS, 16), jnp.float32),
            pltpu.SemaphoreType.DMA,
            pltpu.SemaphoreType.DMA,
            pltpu.SemaphoreType.DMA,
            pltpu.SemaphoreType.DMA,
        ],
        compiler_params=pltpu.CompilerParams(
            use_tc_tiling_on_sc=False, needs_layout_passes=False),
    )(_edge_body)
    acc = edge_kernel(tbl, s1n, edge_index)

    out = pl.pallas_call(
        _fin_body,
        out_shape=jax.ShapeDtypeStruct((N, FEAT), jnp.float32),
    )(acc)
    return out
